# baseline (speedup 1.0000x reference)
import functools

import jax
import jax.numpy as jnp
from jax import lax
from jax.experimental import pallas as pl
from jax.experimental.pallas import tpu as pltpu
from jax.experimental.pallas import tpu_sc as plsc

B = 16384
ITEM_VOCAB = 1000001
TEXT_VOCAB = 10000
EMB = 32
COLOR_LEN = 16
TITLE_LEN = 32

NC = 2
NS = 16
NW = NC * NS
SCHUNK = 1024
NCHUNK = B // SCHUNK
L = 16
IPW = B // NW
NB = 16
TRB = 512


def _tr_body(in_ref, out_ref):
    out_ref[...] = in_ref[...].T


def _ct_body(ctok_hbm, ttok_hbm, ctab_hbm, ttab_hbm, out_hbm,
             ctab_v, ttab_v, ctok_v, ttok_v, outc_v, outt_v, sem):
    w = lax.axis_index("s") * NC + lax.axis_index("c")
    pltpu.sync_copy(ctab_hbm.at[w], ctab_v)
    pltpu.sync_copy(ttab_hbm.at[w], ttab_v)

    for g in range(NCHUNK):
        s0 = pl.multiple_of(g * SCHUNK, SCHUNK)
        pltpu.sync_copy(ctok_hbm.at[:, pl.ds(s0, SCHUNK)], ctok_v)
        pltpu.sync_copy(ttok_hbm.at[:, pl.ds(s0, SCHUNK)], ttok_v)

        def one_group(i):
            sl = pl.ds(i * L, L)
            p = [plsc.load_gather(ctab_v, [ctok_v[t, sl]]) for t in range(4)]
            for t in range(4, COLOR_LEN):
                p[t % 4] = p[t % 4] + plsc.load_gather(ctab_v, [ctok_v[t, sl]])
            outc_v[sl] = ((p[0] + p[1]) + (p[2] + p[3])) * (1.0 / COLOR_LEN)
            q = [plsc.load_gather(ttab_v, [ttok_v[t, sl]]) for t in range(4)]
            for t in range(4, TITLE_LEN):
                q[t % 4] = q[t % 4] + plsc.load_gather(ttab_v, [ttok_v[t, sl]])
            outt_v[sl] = ((q[0] + q[1]) + (q[2] + q[3])) * (1.0 / TITLE_LEN)

        def group_body(i, carry):
            one_group(2 * i)
            one_group(2 * i + 1)
            return carry
        lax.fori_loop(0, SCHUNK // L // 2, group_body, 0)

        pltpu.sync_copy(outc_v, out_hbm.at[w, pl.ds(s0, SCHUNK)])
        pltpu.sync_copy(outt_v, out_hbm.at[EMB + w, pl.ds(s0, SCHUNK)])


def _id_body(ids_hbm, tab_hbm, out_hbm, ids_v, bufs_v, obuf_v, sems):
    w = lax.axis_index("s") * NC + lax.axis_index("c")
    base = pl.multiple_of(w * IPW, IPW)
    pltpu.sync_copy(ids_hbm.at[pl.ds(base, IPW)], ids_v)
    ngroups = IPW // NB

    def issue(g, h):
        idsvec = ids_v[pl.ds(g * NB, NB)]
        for b in range(NB):
            blk = pl.multiple_of((idsvec[b] // 8) * 8, 8)
            pltpu.async_copy(tab_hbm.at[pl.ds(blk, 8), :],
                             bufs_v[h * NB + b].at[:, :], sems[h])

    def process(g, h):
        j0 = g * NB
        idsvec = ids_v[pl.ds(j0, NB)]
        for b in range(NB):
            pltpu.make_async_copy(tab_hbm.at[pl.ds(0, 8), :],
                                  bufs_v[h * NB + b], sems[h]).wait()
            r = idsvec[b] % 8
            for hh in range(EMB // L):
                sl = pl.ds(hh * L, L)
                obuf_v[j0 + b, sl] = bufs_v[h * NB + b][r, sl]

    issue(0, 0)

    def group_body(k, carry):
        g = 2 * k
        issue(g + 1, 1)
        process(g, 0)

        @pl.when(g + 2 < ngroups)
        def _():
            issue(g + 2, 0)
        process(g + 1, 1)
        return carry
    lax.fori_loop(0, ngroups // 2, group_body, 0)

    pltpu.sync_copy(obuf_v, out_hbm.at[pl.ds(base, IPW)])


@jax.jit
def _sc_call(item_id, color_tokens, title_tokens, id_table,
             color_table, title_table):
    mesh = plsc.VectorSubcoreMesh(core_axis_name="c", subcore_axis_name="s")

    ct = functools.partial(
        pl.kernel,
        out_type=jax.ShapeDtypeStruct((2 * EMB, B), jnp.float32),
        mesh=mesh,
        scratch_types=[
            pltpu.VMEM((TEXT_VOCAB,), jnp.float32),
            pltpu.VMEM((TEXT_VOCAB,), jnp.float32),
            pltpu.VMEM((COLOR_LEN, SCHUNK), jnp.int32),
            pltpu.VMEM((TITLE_LEN, SCHUNK), jnp.int32),
            pltpu.VMEM((SCHUNK,), jnp.float32),
            pltpu.VMEM((SCHUNK,), jnp.float32),
            pltpu.SemaphoreType.DMA,
        ],
        compiler_params=pltpu.CompilerParams(use_tc_tiling_on_sc=False,
                                             needs_layout_passes=False),
    )(_ct_body)
    ct_out = ct(color_tokens.T, title_tokens.T, color_table.T, title_table.T)

    nblk = (ITEM_VOCAB + TRB - 1) // TRB
    id_rm = pl.pallas_call(
        _tr_body,
        grid=(nblk,),
        in_specs=[pl.BlockSpec((EMB, TRB), lambda k: (0, k))],
        out_specs=pl.BlockSpec((TRB, EMB), lambda k: (k, 0)),
        out_shape=jax.ShapeDtypeStruct((ITEM_VOCAB, EMB), jnp.float32),
    )(id_table.T)

    idk = functools.partial(
        pl.kernel,
        out_type=jax.ShapeDtypeStruct((B, EMB), jnp.float32),
        mesh=mesh,
        scratch_types=[
            pltpu.VMEM((IPW,), jnp.int32),
            [pltpu.VMEM((8, EMB), jnp.float32) for _ in range(2 * NB)],
            pltpu.VMEM((IPW, EMB), jnp.float32),
            [pltpu.SemaphoreType.DMA for _ in range(2)],
        ],
        compiler_params=pltpu.CompilerParams(use_tc_tiling_on_sc=True,
                                             needs_layout_passes=False),
    )(_id_body)
    id_out = idk(item_id, id_rm)

    return jnp.concatenate([id_out, ct_out.T], axis=1)


def kernel(item_id, color_tokens, title_tokens, id_table, color_table, title_table):
    return _sc_call(item_id, color_tokens, title_tokens,
                    id_table, color_table, title_table)


# speedup vs baseline: 3.4745x; 3.4745x over previous
import functools

import jax
import jax.numpy as jnp
from jax import lax
from jax.experimental import pallas as pl
from jax.experimental.pallas import tpu as pltpu
from jax.experimental.pallas import tpu_sc as plsc

B = 16384
ITEM_VOCAB = 1000001
TEXT_VOCAB = 10000
EMB = 32
COLOR_LEN = 16
TITLE_LEN = 32

NC = 2
NS = 16
NW = NC * NS
SCHUNK = 1024
NCHUNK = B // SCHUNK
L = 16
IPW = B // NW
NB = 16
TRB = 8192


def _tr_body(in_ref, out_ref):
    out_ref[...] = in_ref[...].T


def _ct_body(ctok_hbm, ttok_hbm, ctab_hbm, ttab_hbm, out_hbm,
             ctab_v, ttab_v, ctok_v, ttok_v, outc_v, outt_v, sem):
    w = lax.axis_index("s") * NC + lax.axis_index("c")
    pltpu.sync_copy(ctab_hbm.at[w], ctab_v)
    pltpu.sync_copy(ttab_hbm.at[w], ttab_v)

    for g in range(NCHUNK):
        s0 = pl.multiple_of(g * SCHUNK, SCHUNK)
        pltpu.sync_copy(ctok_hbm.at[:, pl.ds(s0, SCHUNK)], ctok_v)
        pltpu.sync_copy(ttok_hbm.at[:, pl.ds(s0, SCHUNK)], ttok_v)

        def one_group(i):
            sl = pl.ds(i * L, L)
            p = [plsc.load_gather(ctab_v, [ctok_v[t, sl]]) for t in range(4)]
            for t in range(4, COLOR_LEN):
                p[t % 4] = p[t % 4] + plsc.load_gather(ctab_v, [ctok_v[t, sl]])
            outc_v[sl] = ((p[0] + p[1]) + (p[2] + p[3])) * (1.0 / COLOR_LEN)
            q = [plsc.load_gather(ttab_v, [ttok_v[t, sl]]) for t in range(4)]
            for t in range(4, TITLE_LEN):
                q[t % 4] = q[t % 4] + plsc.load_gather(ttab_v, [ttok_v[t, sl]])
            outt_v[sl] = ((q[0] + q[1]) + (q[2] + q[3])) * (1.0 / TITLE_LEN)

        def group_body(i, carry):
            one_group(2 * i)
            one_group(2 * i + 1)
            return carry
        lax.fori_loop(0, SCHUNK // L // 2, group_body, 0)

        pltpu.sync_copy(outc_v, out_hbm.at[w, pl.ds(s0, SCHUNK)])
        pltpu.sync_copy(outt_v, out_hbm.at[EMB + w, pl.ds(s0, SCHUNK)])


def _id_body(ids_hbm, tab_hbm, out_hbm, ids_v, bufs_v, obuf_v, sems):
    w = lax.axis_index("s") * NC + lax.axis_index("c")
    base = pl.multiple_of(w * IPW, IPW)
    pltpu.sync_copy(ids_hbm.at[pl.ds(base, IPW)], ids_v)
    ngroups = IPW // NB

    def issue(g, h):
        idsvec = ids_v[pl.ds(g * NB, NB)]
        for b in range(NB):
            blk = pl.multiple_of((idsvec[b] // 8) * 8, 8)
            pltpu.async_copy(tab_hbm.at[pl.ds(blk, 8), :],
                             bufs_v[h * NB + b].at[:, :], sems[h])

    def process(g, h):
        j0 = g * NB
        idsvec = ids_v[pl.ds(j0, NB)]
        for b in range(NB):
            pltpu.make_async_copy(tab_hbm.at[pl.ds(0, 8), :],
                                  bufs_v[h * NB + b], sems[h]).wait()
            r = idsvec[b] % 8
            for hh in range(EMB // L):
                sl = pl.ds(hh * L, L)
                obuf_v[j0 + b, sl] = bufs_v[h * NB + b][r, sl]

    issue(0, 0)

    def group_body(k, carry):
        g = 2 * k
        issue(g + 1, 1)
        process(g, 0)

        @pl.when(g + 2 < ngroups)
        def _():
            issue(g + 2, 0)
        process(g + 1, 1)
        return carry
    lax.fori_loop(0, ngroups // 2, group_body, 0)

    pltpu.sync_copy(obuf_v, out_hbm.at[pl.ds(base, IPW)])


@jax.jit
def _sc_call(item_id, color_tokens, title_tokens, id_table,
             color_table, title_table):
    mesh = plsc.VectorSubcoreMesh(core_axis_name="c", subcore_axis_name="s")

    ct = functools.partial(
        pl.kernel,
        out_type=jax.ShapeDtypeStruct((2 * EMB, B), jnp.float32),
        mesh=mesh,
        scratch_types=[
            pltpu.VMEM((TEXT_VOCAB,), jnp.float32),
            pltpu.VMEM((TEXT_VOCAB,), jnp.float32),
            pltpu.VMEM((COLOR_LEN, SCHUNK), jnp.int32),
            pltpu.VMEM((TITLE_LEN, SCHUNK), jnp.int32),
            pltpu.VMEM((SCHUNK,), jnp.float32),
            pltpu.VMEM((SCHUNK,), jnp.float32),
            pltpu.SemaphoreType.DMA,
        ],
        compiler_params=pltpu.CompilerParams(use_tc_tiling_on_sc=False,
                                             needs_layout_passes=False),
    )(_ct_body)
    ct_out = ct(color_tokens.T, title_tokens.T, color_table.T, title_table.T)

    nblk = (ITEM_VOCAB + TRB - 1) // TRB
    id_rm = pl.pallas_call(
        _tr_body,
        grid=(nblk,),
        in_specs=[pl.BlockSpec((EMB, TRB), lambda k: (0, k))],
        out_specs=pl.BlockSpec((TRB, EMB), lambda k: (k, 0)),
        out_shape=jax.ShapeDtypeStruct((ITEM_VOCAB, EMB), jnp.float32),
    )(id_table.T)

    idk = functools.partial(
        pl.kernel,
        out_type=jax.ShapeDtypeStruct((B, EMB), jnp.float32),
        mesh=mesh,
        scratch_types=[
            pltpu.VMEM((IPW,), jnp.int32),
            [pltpu.VMEM((8, EMB), jnp.float32) for _ in range(2 * NB)],
            pltpu.VMEM((IPW, EMB), jnp.float32),
            [pltpu.SemaphoreType.DMA for _ in range(2)],
        ],
        compiler_params=pltpu.CompilerParams(use_tc_tiling_on_sc=True,
                                             needs_layout_passes=False),
    )(_id_body)
    id_out = idk(item_id, id_rm)

    return jnp.concatenate([id_out, ct_out.T], axis=1)


def kernel(item_id, color_tokens, title_tokens, id_table, color_table, title_table):
    return _sc_call(item_id, color_tokens, title_tokens,
                    id_table, color_table, title_table)


# speedup vs baseline: 3.6596x; 1.0533x over previous
import functools

import jax
import jax.numpy as jnp
from jax import lax
from jax.experimental import pallas as pl
from jax.experimental.pallas import tpu as pltpu
from jax.experimental.pallas import tpu_sc as plsc

B = 16384
ITEM_VOCAB = 1000001
TEXT_VOCAB = 10000
EMB = 32
COLOR_LEN = 16
TITLE_LEN = 32

NC = 2
NS = 16
NW = NC * NS
SCHUNK = 1024
NCHUNK = B // SCHUNK
L = 16
IPW = B // NW
NB = 16
TRB = 32768


def _tr_body(in_ref, out_ref):
    out_ref[...] = in_ref[...].T


def _ct_body(ctok_hbm, ttok_hbm, ctab_hbm, ttab_hbm, out_hbm,
             ctab_v, ttab_v, ctok_v, ttok_v, outc_v, outt_v, sem):
    w = lax.axis_index("s") * NC + lax.axis_index("c")
    pltpu.sync_copy(ctab_hbm.at[w], ctab_v)
    pltpu.sync_copy(ttab_hbm.at[w], ttab_v)

    for g in range(NCHUNK):
        s0 = pl.multiple_of(g * SCHUNK, SCHUNK)
        pltpu.sync_copy(ctok_hbm.at[:, pl.ds(s0, SCHUNK)], ctok_v)
        pltpu.sync_copy(ttok_hbm.at[:, pl.ds(s0, SCHUNK)], ttok_v)

        def one_group(i):
            sl = pl.ds(i * L, L)
            p = [plsc.load_gather(ctab_v, [ctok_v[t, sl]]) for t in range(4)]
            for t in range(4, COLOR_LEN):
                p[t % 4] = p[t % 4] + plsc.load_gather(ctab_v, [ctok_v[t, sl]])
            outc_v[sl] = ((p[0] + p[1]) + (p[2] + p[3])) * (1.0 / COLOR_LEN)
            q = [plsc.load_gather(ttab_v, [ttok_v[t, sl]]) for t in range(4)]
            for t in range(4, TITLE_LEN):
                q[t % 4] = q[t % 4] + plsc.load_gather(ttab_v, [ttok_v[t, sl]])
            outt_v[sl] = ((q[0] + q[1]) + (q[2] + q[3])) * (1.0 / TITLE_LEN)

        def group_body(i, carry):
            one_group(2 * i)
            one_group(2 * i + 1)
            return carry
        lax.fori_loop(0, SCHUNK // L // 2, group_body, 0)

        pltpu.sync_copy(outc_v, out_hbm.at[w, pl.ds(s0, SCHUNK)])
        pltpu.sync_copy(outt_v, out_hbm.at[EMB + w, pl.ds(s0, SCHUNK)])


def _id_body(ids_hbm, tab_hbm, out_hbm, ids_v, bufs_v, obuf_v, sems):
    w = lax.axis_index("s") * NC + lax.axis_index("c")
    base = pl.multiple_of(w * IPW, IPW)
    pltpu.sync_copy(ids_hbm.at[pl.ds(base, IPW)], ids_v)
    ngroups = IPW // NB

    def issue(g, h):
        idsvec = ids_v[pl.ds(g * NB, NB)]
        for b in range(NB):
            blk = pl.multiple_of((idsvec[b] // 8) * 8, 8)
            pltpu.async_copy(tab_hbm.at[pl.ds(blk, 8), :],
                             bufs_v[h * NB + b].at[:, :], sems[h])

    def process(g, h):
        j0 = g * NB
        idsvec = ids_v[pl.ds(j0, NB)]
        for b in range(NB):
            pltpu.make_async_copy(tab_hbm.at[pl.ds(0, 8), :],
                                  bufs_v[h * NB + b], sems[h]).wait()
            r = idsvec[b] % 8
            for hh in range(EMB // L):
                sl = pl.ds(hh * L, L)
                obuf_v[j0 + b, sl] = bufs_v[h * NB + b][r, sl]

    issue(0, 0)

    def group_body(k, carry):
        g = 2 * k
        issue(g + 1, 1)
        process(g, 0)

        @pl.when(g + 2 < ngroups)
        def _():
            issue(g + 2, 0)
        process(g + 1, 1)
        return carry
    lax.fori_loop(0, ngroups // 2, group_body, 0)

    pltpu.sync_copy(obuf_v, out_hbm.at[pl.ds(base, IPW)])


@jax.jit
def _sc_call(item_id, color_tokens, title_tokens, id_table,
             color_table, title_table):
    mesh = plsc.VectorSubcoreMesh(core_axis_name="c", subcore_axis_name="s")

    ct = functools.partial(
        pl.kernel,
        out_type=jax.ShapeDtypeStruct((2 * EMB, B), jnp.float32),
        mesh=mesh,
        scratch_types=[
            pltpu.VMEM((TEXT_VOCAB,), jnp.float32),
            pltpu.VMEM((TEXT_VOCAB,), jnp.float32),
            pltpu.VMEM((COLOR_LEN, SCHUNK), jnp.int32),
            pltpu.VMEM((TITLE_LEN, SCHUNK), jnp.int32),
            pltpu.VMEM((SCHUNK,), jnp.float32),
            pltpu.VMEM((SCHUNK,), jnp.float32),
            pltpu.SemaphoreType.DMA,
        ],
        compiler_params=pltpu.CompilerParams(use_tc_tiling_on_sc=False,
                                             needs_layout_passes=False),
    )(_ct_body)
    ct_out = ct(color_tokens.T, title_tokens.T, color_table.T, title_table.T)

    nblk = (ITEM_VOCAB + TRB - 1) // TRB
    id_rm = pl.pallas_call(
        _tr_body,
        grid=(nblk,),
        in_specs=[pl.BlockSpec((EMB, TRB), lambda k: (0, k))],
        out_specs=pl.BlockSpec((TRB, EMB), lambda k: (k, 0)),
        out_shape=jax.ShapeDtypeStruct((ITEM_VOCAB, EMB), jnp.float32),
    )(id_table.T)

    idk = functools.partial(
        pl.kernel,
        out_type=jax.ShapeDtypeStruct((B, EMB), jnp.float32),
        mesh=mesh,
        scratch_types=[
            pltpu.VMEM((IPW,), jnp.int32),
            [pltpu.VMEM((8, EMB), jnp.float32) for _ in range(2 * NB)],
            pltpu.VMEM((IPW, EMB), jnp.float32),
            [pltpu.SemaphoreType.DMA for _ in range(2)],
        ],
        compiler_params=pltpu.CompilerParams(use_tc_tiling_on_sc=True,
                                             needs_layout_passes=False),
    )(_id_body)
    id_out = idk(item_id, id_rm)

    return jnp.concatenate([id_out, ct_out.T], axis=1)


def kernel(item_id, color_tokens, title_tokens, id_table, color_table, title_table):
    return _sc_call(item_id, color_tokens, title_tokens,
                    id_table, color_table, title_table)


# speedup vs baseline: 3.6943x; 1.0095x over previous
import functools

import jax
import jax.numpy as jnp
from jax import lax
from jax.experimental import pallas as pl
from jax.experimental.pallas import tpu as pltpu
from jax.experimental.pallas import tpu_sc as plsc

B = 16384
ITEM_VOCAB = 1000001
TEXT_VOCAB = 10000
EMB = 32
COLOR_LEN = 16
TITLE_LEN = 32

NC = 2
NS = 16
NW = NC * NS
SCHUNK = 1024
NCHUNK = B // SCHUNK
L = 16
IPW = B // NW
NB = 16
TRB = 32768


def _tr_body(in_ref, out_ref):
    out_ref[...] = in_ref[...].T


def _ct_body(ctok_hbm, ttok_hbm, ctab_hbm, ttab_hbm, out_hbm,
             ctab_v, ttab_v, ctok_v, ttok_v, outc_v, outt_v, sem):
    w = lax.axis_index("s") * NC + lax.axis_index("c")
    pltpu.sync_copy(ctab_hbm.at[w], ctab_v)
    pltpu.sync_copy(ttab_hbm.at[w], ttab_v)

    for g in range(NCHUNK):
        s0 = pl.multiple_of(g * SCHUNK, SCHUNK)
        pltpu.sync_copy(ctok_hbm.at[:, pl.ds(s0, SCHUNK)], ctok_v)
        pltpu.sync_copy(ttok_hbm.at[:, pl.ds(s0, SCHUNK)], ttok_v)

        def one_group(i):
            sl = pl.ds(i * L, L)
            p = [plsc.load_gather(ctab_v, [ctok_v[t, sl]]) for t in range(4)]
            for t in range(4, COLOR_LEN):
                p[t % 4] = p[t % 4] + plsc.load_gather(ctab_v, [ctok_v[t, sl]])
            outc_v[sl] = ((p[0] + p[1]) + (p[2] + p[3])) * (1.0 / COLOR_LEN)
            q = [plsc.load_gather(ttab_v, [ttok_v[t, sl]]) for t in range(4)]
            for t in range(4, TITLE_LEN):
                q[t % 4] = q[t % 4] + plsc.load_gather(ttab_v, [ttok_v[t, sl]])
            outt_v[sl] = ((q[0] + q[1]) + (q[2] + q[3])) * (1.0 / TITLE_LEN)

        @plsc.parallel_loop(0, SCHUNK // L, unroll=4)
        def _(i):
            one_group(i)

        pltpu.sync_copy(outc_v, out_hbm.at[w, pl.ds(s0, SCHUNK)])
        pltpu.sync_copy(outt_v, out_hbm.at[EMB + w, pl.ds(s0, SCHUNK)])


def _id_body(ids_hbm, tab_hbm, out_hbm, ids_v, bufs_v, obuf_v, sems):
    w = lax.axis_index("s") * NC + lax.axis_index("c")
    base = pl.multiple_of(w * IPW, IPW)
    pltpu.sync_copy(ids_hbm.at[pl.ds(base, IPW)], ids_v)
    ngroups = IPW // NB

    def issue(g, h):
        idsvec = ids_v[pl.ds(g * NB, NB)]
        for b in range(NB):
            blk = pl.multiple_of((idsvec[b] // 8) * 8, 8)
            pltpu.async_copy(tab_hbm.at[pl.ds(blk, 8), :],
                             bufs_v[h * NB + b].at[:, :], sems[h])

    def process(g, h):
        j0 = g * NB
        idsvec = ids_v[pl.ds(j0, NB)]
        for b in range(NB):
            pltpu.make_async_copy(tab_hbm.at[pl.ds(0, 8), :],
                                  bufs_v[h * NB + b], sems[h]).wait()
            r = idsvec[b] % 8
            for hh in range(EMB // L):
                sl = pl.ds(hh * L, L)
                obuf_v[j0 + b, sl] = bufs_v[h * NB + b][r, sl]

    issue(0, 0)

    def group_body(k, carry):
        g = 2 * k
        issue(g + 1, 1)
        process(g, 0)

        @pl.when(g + 2 < ngroups)
        def _():
            issue(g + 2, 0)
        process(g + 1, 1)
        return carry
    lax.fori_loop(0, ngroups // 2, group_body, 0)

    pltpu.sync_copy(obuf_v, out_hbm.at[pl.ds(base, IPW)])


@jax.jit
def _sc_call(item_id, color_tokens, title_tokens, id_table,
             color_table, title_table):
    mesh = plsc.VectorSubcoreMesh(core_axis_name="c", subcore_axis_name="s")

    ct = functools.partial(
        pl.kernel,
        out_type=jax.ShapeDtypeStruct((2 * EMB, B), jnp.float32),
        mesh=mesh,
        scratch_types=[
            pltpu.VMEM((TEXT_VOCAB,), jnp.float32),
            pltpu.VMEM((TEXT_VOCAB,), jnp.float32),
            pltpu.VMEM((COLOR_LEN, SCHUNK), jnp.int32),
            pltpu.VMEM((TITLE_LEN, SCHUNK), jnp.int32),
            pltpu.VMEM((SCHUNK,), jnp.float32),
            pltpu.VMEM((SCHUNK,), jnp.float32),
            pltpu.SemaphoreType.DMA,
        ],
        compiler_params=pltpu.CompilerParams(use_tc_tiling_on_sc=False,
                                             needs_layout_passes=False),
    )(_ct_body)
    ct_out = ct(color_tokens.T, title_tokens.T, color_table.T, title_table.T)

    nblk = (ITEM_VOCAB + TRB - 1) // TRB
    id_rm = pl.pallas_call(
        _tr_body,
        grid=(nblk,),
        in_specs=[pl.BlockSpec((EMB, TRB), lambda k: (0, k))],
        out_specs=pl.BlockSpec((TRB, EMB), lambda k: (k, 0)),
        out_shape=jax.ShapeDtypeStruct((ITEM_VOCAB, EMB), jnp.float32),
    )(id_table.T)

    idk = functools.partial(
        pl.kernel,
        out_type=jax.ShapeDtypeStruct((B, EMB), jnp.float32),
        mesh=mesh,
        scratch_types=[
            pltpu.VMEM((IPW,), jnp.int32),
            [pltpu.VMEM((8, EMB), jnp.float32) for _ in range(2 * NB)],
            pltpu.VMEM((IPW, EMB), jnp.float32),
            [pltpu.SemaphoreType.DMA for _ in range(2)],
        ],
        compiler_params=pltpu.CompilerParams(use_tc_tiling_on_sc=True,
                                             needs_layout_passes=False),
    )(_id_body)
    id_out = idk(item_id, id_rm)

    return jnp.concatenate([id_out, ct_out.T], axis=1)


def kernel(item_id, color_tokens, title_tokens, id_table, color_table, title_table):
    return _sc_call(item_id, color_tokens, title_tokens,
                    id_table, color_table, title_table)


# speedup vs baseline: 3.9124x; 1.0590x over previous
import functools

import jax
import jax.numpy as jnp
from jax import lax
from jax.experimental import pallas as pl
from jax.experimental.pallas import tpu as pltpu
from jax.experimental.pallas import tpu_sc as plsc

B = 16384
ITEM_VOCAB = 1000001
TEXT_VOCAB = 10000
EMB = 32
COLOR_LEN = 16
TITLE_LEN = 32

NC = 2
NS = 16
NW = NC * NS
SCHUNK = 1024
NCHUNK = B // SCHUNK
L = 16
IPW = B // NW
NB = 16
TRB = 8192
NPK = 262144


def _tr_body(i0, i1, i2, i3, out_ref):
    out_ref[...] = jnp.concatenate(
        [i0[...].T, i1[...].T, i2[...].T, i3[...].T], axis=1)


def _ct_body(ctok_hbm, ttok_hbm, ctab_hbm, ttab_hbm, out_hbm,
             ctab_v, ttab_v, ctok_v, ttok_v, outc_v, outt_v, sem):
    w = lax.axis_index("s") * NC + lax.axis_index("c")
    pltpu.sync_copy(ctab_hbm.at[w], ctab_v)
    pltpu.sync_copy(ttab_hbm.at[w], ttab_v)

    for g in range(NCHUNK):
        s0 = pl.multiple_of(g * SCHUNK, SCHUNK)
        pltpu.sync_copy(ctok_hbm.at[:, pl.ds(s0, SCHUNK)], ctok_v)
        pltpu.sync_copy(ttok_hbm.at[:, pl.ds(s0, SCHUNK)], ttok_v)

        def one_group(i):
            sl = pl.ds(i * L, L)
            p = [plsc.load_gather(ctab_v, [ctok_v[t, sl]]) for t in range(4)]
            for t in range(4, COLOR_LEN):
                p[t % 4] = p[t % 4] + plsc.load_gather(ctab_v, [ctok_v[t, sl]])
            outc_v[sl] = ((p[0] + p[1]) + (p[2] + p[3])) * (1.0 / COLOR_LEN)
            q = [plsc.load_gather(ttab_v, [ttok_v[t, sl]]) for t in range(4)]
            for t in range(4, TITLE_LEN):
                q[t % 4] = q[t % 4] + plsc.load_gather(ttab_v, [ttok_v[t, sl]])
            outt_v[sl] = ((q[0] + q[1]) + (q[2] + q[3])) * (1.0 / TITLE_LEN)

        @plsc.parallel_loop(0, SCHUNK // L, unroll=4)
        def _(i):
            one_group(i)

        pltpu.sync_copy(outc_v, out_hbm.at[w, pl.ds(s0, SCHUNK)])
        pltpu.sync_copy(outt_v, out_hbm.at[EMB + w, pl.ds(s0, SCHUNK)])


def _id_body(ids_hbm, tab_hbm, out_hbm, ids_v, bufs_v, obuf_v, sems):
    w = lax.axis_index("s") * NC + lax.axis_index("c")
    base = pl.multiple_of(w * IPW, IPW)
    pltpu.sync_copy(ids_hbm.at[pl.ds(base, IPW)], ids_v)
    ngroups = IPW // NB

    def issue(g, h):
        idsvec = ids_v[pl.ds(g * NB, NB)]
        for b in range(NB):
            row = idsvec[b] % NPK
            blk = pl.multiple_of((row // 8) * 8, 8)
            pltpu.async_copy(tab_hbm.at[pl.ds(blk, 8), :],
                             bufs_v[h * NB + b].at[:, :], sems[h])

    def process(g, h):
        j0 = g * NB
        idsvec = ids_v[pl.ds(j0, NB)]
        for b in range(NB):
            pltpu.make_async_copy(tab_hbm.at[pl.ds(0, 8), :],
                                  bufs_v[h * NB + b], sems[h]).wait()
            r = (idsvec[b] % NPK) % 8
            grp = idsvec[b] // NPK
            for hh in range(EMB // L):
                sl = pl.ds(hh * L, L)
                cands = [bufs_v[h * NB + b][r, pl.ds(jj * EMB + hh * L, L)]
                         for jj in range(4)]
                lo = jnp.where(grp % 2 == 1, cands[1], cands[0])
                hi = jnp.where(grp % 2 == 1, cands[3], cands[2])
                obuf_v[j0 + b, sl] = jnp.where(grp >= 2, hi, lo)

    issue(0, 0)

    def group_body(k, carry):
        g = 2 * k
        issue(g + 1, 1)
        process(g, 0)

        @pl.when(g + 2 < ngroups)
        def _():
            issue(g + 2, 0)
        process(g + 1, 1)
        return carry
    lax.fori_loop(0, ngroups // 2, group_body, 0)

    pltpu.sync_copy(obuf_v, out_hbm.at[pl.ds(base, IPW)])


@jax.jit
def _sc_call(item_id, color_tokens, title_tokens, id_table,
             color_table, title_table):
    mesh = plsc.VectorSubcoreMesh(core_axis_name="c", subcore_axis_name="s")

    ct = functools.partial(
        pl.kernel,
        out_type=jax.ShapeDtypeStruct((2 * EMB, B), jnp.float32),
        mesh=mesh,
        scratch_types=[
            pltpu.VMEM((TEXT_VOCAB,), jnp.float32),
            pltpu.VMEM((TEXT_VOCAB,), jnp.float32),
            pltpu.VMEM((COLOR_LEN, SCHUNK), jnp.int32),
            pltpu.VMEM((TITLE_LEN, SCHUNK), jnp.int32),
            pltpu.VMEM((SCHUNK,), jnp.float32),
            pltpu.VMEM((SCHUNK,), jnp.float32),
            pltpu.SemaphoreType.DMA,
        ],
        compiler_params=pltpu.CompilerParams(use_tc_tiling_on_sc=False,
                                             needs_layout_passes=False),
    )(_ct_body)
    ct_out = ct(color_tokens.T, title_tokens.T, color_table.T, title_table.T)

    nblk = NPK // TRB
    tt = id_table.T
    id_rm = pl.pallas_call(
        _tr_body,
        grid=(nblk,),
        in_specs=[pl.BlockSpec(
            (EMB, TRB),
            lambda k, jj=jj: (0, jnp.minimum(k + jj * nblk,
                                             (ITEM_VOCAB - 1) // TRB)))
                  for jj in range(4)],
        out_specs=pl.BlockSpec((TRB, 4 * EMB), lambda k: (k, 0)),
        out_shape=jax.ShapeDtypeStruct((NPK, 4 * EMB), jnp.float32),
    )(tt, tt, tt, tt)

    idk = functools.partial(
        pl.kernel,
        out_type=jax.ShapeDtypeStruct((B, EMB), jnp.float32),
        mesh=mesh,
        scratch_types=[
            pltpu.VMEM((IPW,), jnp.int32),
            [pltpu.VMEM((8, 4 * EMB), jnp.float32) for _ in range(2 * NB)],
            pltpu.VMEM((IPW, EMB), jnp.float32),
            [pltpu.SemaphoreType.DMA for _ in range(2)],
        ],
        compiler_params=pltpu.CompilerParams(use_tc_tiling_on_sc=True,
                                             needs_layout_passes=False),
    )(_id_body)
    id_out = idk(item_id, id_rm)

    return jnp.concatenate([id_out, ct_out.T], axis=1)


def kernel(item_id, color_tokens, title_tokens, id_table, color_table, title_table):
    return _sc_call(item_id, color_tokens, title_tokens,
                    id_table, color_table, title_table)
